# baseline (speedup 1.0000x reference)
import jax
import jax.numpy as jnp
from jax.experimental import pallas as pl

_B, _T, _D = 4, 2048, 128
_F, _NB, _ES, _NE = 32, 4, 2, 8
_DFE, _DRH, _DEH = 64, 64, 64
_BF = _F // _NB
_TS = 1024


def _moe_kernel(x_ref, f_ref, m_ref, v_ref, g_ref, b_ref, sfw_ref, sfb_ref,
                bpw_ref, bpb_ref, brw1_ref, brb1_ref, brw2_ref, brb2_ref,
                irw1_ref, irb1_ref, irw2_ref, irb2_ref,
                w1_ref, b1e_ref, w2_ref, eb2_ref, exp_ref, alpha_ref,
                nh_ref, gw_ref, gl_ref, bw_ref, blg_ref, sd_ref):
    x = x_ref[...]
    f = f_ref[...]

    hn = (x - m_ref[...]) / jnp.sqrt(v_ref[...] + 1e-5) * g_ref[...] + b_ref[...]

    sfe = jnp.dot(f, sfw_ref[...]) + sfb_ref[...]
    br_in = jnp.concatenate([hn, sfe], axis=1)
    bh = jax.nn.gelu(jnp.dot(br_in, brw1_ref[...]) + brb1_ref[...])
    bl = jnp.dot(bh, brw2_ref[...]) + brb2_ref[...]

    m1 = jnp.max(bl, axis=1, keepdims=True)
    nmax = jnp.sum((bl == m1).astype(jnp.float32), axis=1, keepdims=True)
    runner = jnp.max(jnp.where(bl < m1, bl, -jnp.inf), axis=1, keepdims=True)
    thr = jnp.where(nmax > 1.0, m1, runner)
    be = jnp.exp(jnp.where(bl >= thr, bl, -1e9) - m1)
    bw = be / jnp.sum(be, axis=1, keepdims=True)

    gw_parts, gl_parts = [], []
    for bi in range(_NB):
        bfe = (jnp.dot(f[:, _BF * bi:_BF * (bi + 1)],
                       bpw_ref[_BF * bi:_BF * (bi + 1), :])
               + bpb_ref[bi:bi + 1, :])
        iin = jnp.concatenate([hn, bfe], axis=1)
        ih = jax.nn.gelu(jnp.dot(iin, irw1_ref[192 * bi:192 * (bi + 1), :])
                         + irb1_ref[bi:bi + 1, :])
        il = jnp.dot(ih, irw2_ref[_DRH * bi:_DRH * (bi + 1), :]) + irb2_ref[bi:bi + 1, :]
        im = jnp.max(il, axis=1, keepdims=True)
        ie = jnp.exp(jnp.where(il >= im, il, -1e9) - im)
        iw = ie / jnp.sum(ie, axis=1, keepdims=True)
        gw_parts.append(bw[:, bi:bi + 1] * iw)
        gl_parts.append(bl[:, bi:bi + 1] + il)
    gw = jnp.concatenate(gw_parts, axis=1)
    gl = jnp.concatenate(gl_parts, axis=1)

    h1 = jax.nn.gelu(jnp.dot(hn.astype(jnp.bfloat16), w1_ref[...],
                             preferred_element_type=jnp.float32)
                     + b1e_ref[...])
    grep = jnp.dot(gw, exp_ref[...], preferred_element_type=jnp.float32)
    delta = (jnp.dot((h1 * grep).astype(jnp.bfloat16), w2_ref[...],
                     preferred_element_type=jnp.float32)
             + jnp.dot(gw, eb2_ref[...], preferred_element_type=jnp.float32))

    sd_ref[...] = delta
    nh_ref[...] = x + alpha_ref[0, 0] * delta
    gw_ref[...] = gw
    gl_ref[...] = gl
    bw_ref[...] = bw
    blg_ref[...] = bl


def kernel(hidden, feat, ln_g, ln_b, sfW, sfb, bpW, bpb, brW1, brb1, brW2,
           brb2, irW1, irb1, irW2, irb2, eW1, eb1, eW2, eb2, alpha):
    N = _B * _T
    x = hidden.reshape(N, _D)
    f = feat.reshape(N, _F)

    m3 = hidden.mean(-1, keepdims=True)
    v3 = ((hidden - m3) ** 2).mean(-1, keepdims=True)
    mstat = m3.reshape(N, 1)
    vstat = v3.reshape(N, 1)

    W1s = eW1.transpose(1, 0, 2).reshape(_D, _NE * _DEH).astype(jnp.bfloat16)
    b1e = eb1.reshape(1, _NE * _DEH)
    W2s = eW2.reshape(_NE * _DEH, _D).astype(jnp.bfloat16)
    Exp = jnp.kron(jnp.eye(_NE, dtype=jnp.float32),
                   jnp.ones((1, _DEH), jnp.float32))

    grid = (N // _TS,)
    full = lambda r, c: pl.BlockSpec((r, c), lambda i: (0, 0))
    tok = lambda c: pl.BlockSpec((_TS, c), lambda i: (i, 0))

    out_shape = (
        jax.ShapeDtypeStruct((N, _D), jnp.float32),
        jax.ShapeDtypeStruct((N, _NE), jnp.float32),
        jax.ShapeDtypeStruct((N, _NE), jnp.float32),
        jax.ShapeDtypeStruct((N, _NB), jnp.float32),
        jax.ShapeDtypeStruct((N, _NB), jnp.float32),
        jax.ShapeDtypeStruct((N, _D), jnp.float32),
    )

    nh, gw, gl, bw, blw, sd = pl.pallas_call(
        _moe_kernel,
        grid=grid,
        in_specs=[
            tok(_D), tok(_F), tok(1), tok(1),
            full(1, _D), full(1, _D),
            full(_F, _DFE), full(1, _DFE),
            full(_NB * _BF, _DFE), full(_NB, _DFE),
            full(_D + _DFE, _DRH), full(1, _DRH),
            full(_DRH, _NB), full(1, _NB),
            full(_NB * (_D + _DFE), _DRH), full(_NB, _DRH),
            full(_NB * _DRH, _ES), full(_NB, _ES),
            full(_D, _NE * _DEH), full(1, _NE * _DEH),
            full(_NE * _DEH, _D), full(_NE, _D), full(_NE, _NE * _DEH),
            full(1, 1),
        ],
        out_specs=(tok(_D), tok(_NE), tok(_NE), tok(_NB), tok(_NB), tok(_D)),
        out_shape=out_shape,
    )(x, f, mstat, vstat, ln_g.reshape(1, _D), ln_b.reshape(1, _D),
      sfW, sfb.reshape(1, _DFE),
      bpW.reshape(_NB * _BF, _DFE), bpb,
      brW1, brb1.reshape(1, _DRH), brW2, brb2.reshape(1, _NB),
      irW1.reshape(_NB * (_D + _DFE), _DRH), irb1,
      irW2.reshape(_NB * _DRH, _ES), irb2,
      W1s, b1e, W2s, eb2, Exp, alpha.reshape(1, 1))

    shp3 = lambda a: a.reshape(_B, _T, -1)
    return (shp3(nh), shp3(gw), shp3(gl), shp3(bw), shp3(blw), shp3(sd))


# speedup vs baseline: 1.1367x; 1.1367x over previous
import jax
import jax.numpy as jnp
from jax.experimental import pallas as pl

_B, _T, _D = 4, 2048, 128
_F, _NB, _ES, _NE = 32, 4, 2, 8
_DFE, _DRH, _DEH = 64, 64, 64
_BF = _F // _NB
_TS = 2048


def _moe_kernel(x_ref, f_ref, m_ref, v_ref, g_ref, b_ref, sfw_ref, sfb_ref,
                bpw_ref, bpb_ref, brw1_ref, brb1_ref, brw2_ref, brb2_ref,
                irw1_ref, irb1_ref, irw2_ref, irb2_ref,
                w1_ref, b1e_ref, w2_ref, eb2_ref, exp_ref, alpha_ref,
                nh_ref, gw_ref, gl_ref, bw_ref, blg_ref, sd_ref):
    x = x_ref[...]
    f = f_ref[...]

    hn = (x - m_ref[...]) / jnp.sqrt(v_ref[...] + 1e-5) * g_ref[...] + b_ref[...]

    sfe = jnp.dot(f, sfw_ref[...]) + sfb_ref[...]
    br_in = jnp.concatenate([hn, sfe], axis=1)
    bh = jax.nn.gelu(jnp.dot(br_in, brw1_ref[...]) + brb1_ref[...])
    bl = jnp.dot(bh, brw2_ref[...]) + brb2_ref[...]

    m1 = jnp.max(bl, axis=1, keepdims=True)
    nmax = jnp.sum((bl == m1).astype(jnp.float32), axis=1, keepdims=True)
    runner = jnp.max(jnp.where(bl < m1, bl, -jnp.inf), axis=1, keepdims=True)
    thr = jnp.where(nmax > 1.0, m1, runner)
    be = jnp.exp(jnp.where(bl >= thr, bl, -1e9) - m1)
    bw = be / jnp.sum(be, axis=1, keepdims=True)

    gw_parts, gl_parts = [], []
    for bi in range(_NB):
        bfe = (jnp.dot(f[:, _BF * bi:_BF * (bi + 1)],
                       bpw_ref[_BF * bi:_BF * (bi + 1), :])
               + bpb_ref[bi:bi + 1, :])
        iin = jnp.concatenate([hn, bfe], axis=1)
        ih = jax.nn.gelu(jnp.dot(iin, irw1_ref[192 * bi:192 * (bi + 1), :])
                         + irb1_ref[bi:bi + 1, :])
        il = jnp.dot(ih, irw2_ref[_DRH * bi:_DRH * (bi + 1), :]) + irb2_ref[bi:bi + 1, :]
        im = jnp.max(il, axis=1, keepdims=True)
        ie = jnp.exp(jnp.where(il >= im, il, -1e9) - im)
        iw = ie / jnp.sum(ie, axis=1, keepdims=True)
        gw_parts.append(bw[:, bi:bi + 1] * iw)
        gl_parts.append(bl[:, bi:bi + 1] + il)
    gw = jnp.concatenate(gw_parts, axis=1)
    gl = jnp.concatenate(gl_parts, axis=1)

    h1 = jax.nn.gelu(jnp.dot(hn.astype(jnp.bfloat16), w1_ref[...],
                             preferred_element_type=jnp.float32)
                     + b1e_ref[...])
    grep = jnp.dot(gw, exp_ref[...], preferred_element_type=jnp.float32)
    delta = (jnp.dot((h1 * grep).astype(jnp.bfloat16), w2_ref[...],
                     preferred_element_type=jnp.float32)
             + jnp.dot(gw, eb2_ref[...], preferred_element_type=jnp.float32))

    sd_ref[...] = delta
    nh_ref[...] = x + alpha_ref[0, 0] * delta
    gw_ref[...] = gw
    gl_ref[...] = gl
    bw_ref[...] = bw
    blg_ref[...] = bl


def kernel(hidden, feat, ln_g, ln_b, sfW, sfb, bpW, bpb, brW1, brb1, brW2,
           brb2, irW1, irb1, irW2, irb2, eW1, eb1, eW2, eb2, alpha):
    N = _B * _T
    x = hidden.reshape(N, _D)
    f = feat.reshape(N, _F)

    m3 = hidden.mean(-1, keepdims=True)
    v3 = ((hidden - m3) ** 2).mean(-1, keepdims=True)
    mstat = m3.reshape(N, 1)
    vstat = v3.reshape(N, 1)

    W1s = eW1.transpose(1, 0, 2).reshape(_D, _NE * _DEH).astype(jnp.bfloat16)
    b1e = eb1.reshape(1, _NE * _DEH)
    W2s = eW2.reshape(_NE * _DEH, _D).astype(jnp.bfloat16)
    Exp = jnp.kron(jnp.eye(_NE, dtype=jnp.float32),
                   jnp.ones((1, _DEH), jnp.float32))

    grid = (N // _TS,)
    full = lambda r, c: pl.BlockSpec((r, c), lambda i: (0, 0))
    tok = lambda c: pl.BlockSpec((_TS, c), lambda i: (i, 0))

    out_shape = (
        jax.ShapeDtypeStruct((N, _D), jnp.float32),
        jax.ShapeDtypeStruct((N, _NE), jnp.float32),
        jax.ShapeDtypeStruct((N, _NE), jnp.float32),
        jax.ShapeDtypeStruct((N, _NB), jnp.float32),
        jax.ShapeDtypeStruct((N, _NB), jnp.float32),
        jax.ShapeDtypeStruct((N, _D), jnp.float32),
    )

    nh, gw, gl, bw, blw, sd = pl.pallas_call(
        _moe_kernel,
        grid=grid,
        in_specs=[
            tok(_D), tok(_F), tok(1), tok(1),
            full(1, _D), full(1, _D),
            full(_F, _DFE), full(1, _DFE),
            full(_NB * _BF, _DFE), full(_NB, _DFE),
            full(_D + _DFE, _DRH), full(1, _DRH),
            full(_DRH, _NB), full(1, _NB),
            full(_NB * (_D + _DFE), _DRH), full(_NB, _DRH),
            full(_NB * _DRH, _ES), full(_NB, _ES),
            full(_D, _NE * _DEH), full(1, _NE * _DEH),
            full(_NE * _DEH, _D), full(_NE, _D), full(_NE, _NE * _DEH),
            full(1, 1),
        ],
        out_specs=(tok(_D), tok(_NE), tok(_NE), tok(_NB), tok(_NB), tok(_D)),
        out_shape=out_shape,
    )(x, f, mstat, vstat, ln_g.reshape(1, _D), ln_b.reshape(1, _D),
      sfW, sfb.reshape(1, _DFE),
      bpW.reshape(_NB * _BF, _DFE), bpb,
      brW1, brb1.reshape(1, _DRH), brW2, brb2.reshape(1, _NB),
      irW1.reshape(_NB * (_D + _DFE), _DRH), irb1,
      irW2.reshape(_NB * _DRH, _ES), irb2,
      W1s, b1e, W2s, eb2, Exp, alpha.reshape(1, 1))

    shp3 = lambda a: a.reshape(_B, _T, -1)
    return (shp3(nh), shp3(gw), shp3(gl), shp3(bw), shp3(blw), shp3(sd))
